# baseline (speedup 1.0000x reference)
import functools

import jax
import jax.numpy as jnp
from jax.experimental import pallas as pl
from jax.experimental.pallas import tpu as pltpu

_CHUNKS = ((0, 128), (128, 256), (384, 512), (896, 1024), (1920, 2048),
           (3968, 2560), (6528, 2688))
_MAXROWS = 2688


def _dist_kernel(f_hbm, c_ref, o_hbm, fbuf0, fbuf1, fbuf2, obuf0, obuf1,
                 obuf2, cs_buf, ld_sem, st_sem):
    fbufs = (fbuf0, fbuf1, fbuf2)
    obufs = (obuf0, obuf1, obuf2)

    loads = [
        pltpu.make_async_copy(
            f_hbm.at[pl.ds(r0, sz), :],
            fbufs[i % 3].at[pl.ds(0, sz), :],
            ld_sem.at[i % 3],
        )
        for i, (r0, sz) in enumerate(_CHUNKS)
    ]
    loads[0].start()
    loads[1].start()

    c = c_ref[...]
    cs_buf[...] = (-2.0 * c).astype(jnp.bfloat16)
    c2 = jnp.sum(c * c, axis=1)[None, :]

    stores = []
    for i, (r0, sz) in enumerate(_CHUNKS):
        if i + 2 < len(_CHUNKS):
            loads[i + 2].start()
        loads[i].wait()
        if i >= 3:
            stores[i - 3].wait()
        f = fbufs[i % 3][pl.ds(0, sz), :]
        f2 = jnp.sum(f * f, axis=1, keepdims=True)
        fc = jax.lax.dot_general(
            f.astype(jnp.bfloat16), cs_buf[...],
            (((1,), (1,)), ((), ())),
            preferred_element_type=jnp.float32,
        )
        obufs[i % 3][pl.ds(0, sz), :] = (fc + f2) + c2
        st = pltpu.make_async_copy(
            obufs[i % 3].at[pl.ds(0, sz), :],
            o_hbm.at[pl.ds(r0, sz), :],
            st_sem.at[i % 3],
        )
        st.start()
        stores.append(st)
    stores[-3].wait()
    stores[-2].wait()
    stores[-1].wait()


@jax.jit
def _dists(f, c):
    M, D = f.shape
    K = c.shape[0]
    return pl.pallas_call(
        _dist_kernel,
        in_specs=[
            pl.BlockSpec(memory_space=pl.ANY),
            pl.BlockSpec((K, D), lambda: (0, 0)),
        ],
        out_specs=pl.BlockSpec(memory_space=pl.ANY),
        out_shape=jax.ShapeDtypeStruct((M, K), jnp.float32),
        scratch_shapes=[
            pltpu.VMEM((_MAXROWS, D), jnp.float32),
            pltpu.VMEM((_MAXROWS, D), jnp.float32),
            pltpu.VMEM((_MAXROWS, D), jnp.float32),
            pltpu.VMEM((_MAXROWS, K), jnp.float32),
            pltpu.VMEM((_MAXROWS, K), jnp.float32),
            pltpu.VMEM((_MAXROWS, K), jnp.float32),
            pltpu.VMEM((K, D), jnp.bfloat16),
            pltpu.SemaphoreType.DMA((3,)),
            pltpu.SemaphoreType.DMA((3,)),
        ],
    )(f, c)


def kernel(features, Ck):
    B, S, D = features.shape
    K = Ck.shape[1]
    f = features.reshape(B * S, D)
    c = Ck.reshape(K, D)
    dists = _dists(f, c)
    return dists.reshape(B, S, K)


# speedup vs baseline: 1.0036x; 1.0036x over previous
import functools

import jax
import jax.numpy as jnp
from jax.experimental import pallas as pl
from jax.experimental.pallas import tpu as pltpu

_CHUNKS = ((0, 256), (256, 512), (768, 1024), (1792, 2048), (3840, 2688),
           (6528, 2688))
_MAXROWS = 2688


def _dist_kernel(f_hbm, c_ref, o_hbm, fbuf0, fbuf1, fbuf2, obuf0, obuf1,
                 obuf2, cs_buf, ld_sem, st_sem):
    fbufs = (fbuf0, fbuf1, fbuf2)
    obufs = (obuf0, obuf1, obuf2)

    loads = [
        pltpu.make_async_copy(
            f_hbm.at[pl.ds(r0, sz), :],
            fbufs[i % 3].at[pl.ds(0, sz), :],
            ld_sem.at[i % 3],
        )
        for i, (r0, sz) in enumerate(_CHUNKS)
    ]
    loads[0].start()
    loads[1].start()

    c = c_ref[...]
    cs_buf[...] = (-2.0 * c).astype(jnp.bfloat16)
    c2 = jnp.sum(c * c, axis=1)[None, :]

    stores = []
    for i, (r0, sz) in enumerate(_CHUNKS):
        if i + 2 < len(_CHUNKS):
            loads[i + 2].start()
        loads[i].wait()
        if i >= 3:
            stores[i - 3].wait()
        f = fbufs[i % 3][pl.ds(0, sz), :]
        f2 = jnp.sum(f * f, axis=1, keepdims=True)
        fc = jax.lax.dot_general(
            f.astype(jnp.bfloat16), cs_buf[...],
            (((1,), (1,)), ((), ())),
            preferred_element_type=jnp.float32,
        )
        obufs[i % 3][pl.ds(0, sz), :] = (fc + f2) + c2
        st = pltpu.make_async_copy(
            obufs[i % 3].at[pl.ds(0, sz), :],
            o_hbm.at[pl.ds(r0, sz), :],
            st_sem.at[i % 3],
        )
        st.start()
        stores.append(st)
    stores[-3].wait()
    stores[-2].wait()
    stores[-1].wait()


@jax.jit
def _dists(f, c):
    M, D = f.shape
    K = c.shape[0]
    return pl.pallas_call(
        _dist_kernel,
        in_specs=[
            pl.BlockSpec(memory_space=pl.ANY),
            pl.BlockSpec((K, D), lambda: (0, 0)),
        ],
        out_specs=pl.BlockSpec(memory_space=pl.ANY),
        out_shape=jax.ShapeDtypeStruct((M, K), jnp.float32),
        scratch_shapes=[
            pltpu.VMEM((_MAXROWS, D), jnp.float32),
            pltpu.VMEM((_MAXROWS, D), jnp.float32),
            pltpu.VMEM((_MAXROWS, D), jnp.float32),
            pltpu.VMEM((_MAXROWS, K), jnp.float32),
            pltpu.VMEM((_MAXROWS, K), jnp.float32),
            pltpu.VMEM((_MAXROWS, K), jnp.float32),
            pltpu.VMEM((K, D), jnp.bfloat16),
            pltpu.SemaphoreType.DMA((3,)),
            pltpu.SemaphoreType.DMA((3,)),
        ],
    )(f, c)


def kernel(features, Ck):
    B, S, D = features.shape
    K = Ck.shape[1]
    f = features.reshape(B * S, D)
    c = Ck.reshape(K, D)
    dists = _dists(f, c)
    return dists.reshape(B, S, K)


# speedup vs baseline: 1.0085x; 1.0050x over previous
import functools

import jax
import jax.numpy as jnp
from jax.experimental import pallas as pl
from jax.experimental.pallas import tpu as pltpu

_CHUNKS = ((0, 256), (256, 512), (768, 1024), (1792, 2048), (3840, 2560),
           (6400, 2816))
_MAXROWS = 2816


def _dist_kernel(f_hbm, c_ref, o_hbm, fbuf0, fbuf1, fbuf2, obuf0, obuf1,
                 obuf2, cs_buf, ld_sem, st_sem):
    fbufs = (fbuf0, fbuf1, fbuf2)
    obufs = (obuf0, obuf1, obuf2)

    loads = [
        pltpu.make_async_copy(
            f_hbm.at[pl.ds(r0, sz), :],
            fbufs[i % 3].at[pl.ds(0, sz), :],
            ld_sem.at[i % 3],
        )
        for i, (r0, sz) in enumerate(_CHUNKS)
    ]
    loads[0].start()
    loads[1].start()

    c = c_ref[...]
    cs_buf[...] = (-2.0 * c).astype(jnp.bfloat16)
    c2 = jnp.sum(c * c, axis=1)[None, :]

    stores = []
    for i, (r0, sz) in enumerate(_CHUNKS):
        if i + 2 < len(_CHUNKS):
            loads[i + 2].start()
        loads[i].wait()
        if i >= 3:
            stores[i - 3].wait()
        f = fbufs[i % 3][pl.ds(0, sz), :]
        f2 = jnp.sum(f * f, axis=1, keepdims=True)
        fc = jax.lax.dot_general(
            f.astype(jnp.bfloat16), cs_buf[...],
            (((1,), (1,)), ((), ())),
            preferred_element_type=jnp.float32,
        )
        obufs[i % 3][pl.ds(0, sz), :] = (fc + f2) + c2
        st = pltpu.make_async_copy(
            obufs[i % 3].at[pl.ds(0, sz), :],
            o_hbm.at[pl.ds(r0, sz), :],
            st_sem.at[i % 3],
        )
        st.start()
        stores.append(st)
    stores[-3].wait()
    stores[-2].wait()
    stores[-1].wait()


@jax.jit
def _dists(f, c):
    M, D = f.shape
    K = c.shape[0]
    return pl.pallas_call(
        _dist_kernel,
        in_specs=[
            pl.BlockSpec(memory_space=pl.ANY),
            pl.BlockSpec((K, D), lambda: (0, 0)),
        ],
        out_specs=pl.BlockSpec(memory_space=pl.ANY),
        out_shape=jax.ShapeDtypeStruct((M, K), jnp.float32),
        scratch_shapes=[
            pltpu.VMEM((_MAXROWS, D), jnp.float32),
            pltpu.VMEM((_MAXROWS, D), jnp.float32),
            pltpu.VMEM((_MAXROWS, D), jnp.float32),
            pltpu.VMEM((_MAXROWS, K), jnp.float32),
            pltpu.VMEM((_MAXROWS, K), jnp.float32),
            pltpu.VMEM((_MAXROWS, K), jnp.float32),
            pltpu.VMEM((K, D), jnp.bfloat16),
            pltpu.SemaphoreType.DMA((3,)),
            pltpu.SemaphoreType.DMA((3,)),
        ],
    )(f, c)


def kernel(features, Ck):
    B, S, D = features.shape
    K = Ck.shape[1]
    f = features.reshape(B * S, D)
    c = Ck.reshape(K, D)
    dists = _dists(f, c)
    return dists.reshape(B, S, K)
